# Initial kernel scaffold; baseline (speedup 1.0000x reference)
import jax
import jax.numpy as jnp
from jax.experimental import pallas as pl


def kernel(x, x_emb, edge_index, edge_attr, edge_attr_emb, vert_ff_w, vert_emb_t, edge_ff_w, edge_emb_t, edge_lin_w, conv_rel_w, conv_root_w, conv_b, bn_gamma, bn_beta, res_lin_w, head_w1, head_b1, head_w2, head_b2):
    raise NotImplementedError("write your pallas kernel here")


# speedup vs baseline: 3.0753x; 3.0753x over previous
import functools

import jax
import jax.numpy as jnp
from jax import lax
from jax.experimental import pallas as pl
from jax.experimental.pallas import tpu as pltpu
from jax.experimental.pallas import tpu_sc as plsc

NC = 2
NS = 16
LANES = 16
NW = NC * NS

_N = 10000
_E = 320000
_H = 128
_L = 5
_CH = 128
_NCHUNK = _E // _CH
_JMAX = -(-_NCHUNK // NW)
_ZR = 80
_ZNCHUNK = _N // _ZR
_ZQ = -(-_ZNCHUNK // NS)
_VCH = 80
_VNCHUNK = _N // _VCH
_VJ = -(-_VNCHUNK // NW)
_CW = 32


def _zero_2d(ref, nrows, ncols):
    z = jnp.zeros((LANES,), jnp.float32)

    def body(i, _):
        for g in range(ncols // LANES):
            ref[i, pl.ds(g * LANES, LANES)] = z
        return 0

    lax.fori_loop(0, nrows, body, 0)


def _make_edge_pre():
    mesh = plsc.VectorSubcoreMesh(core_axis_name="c", subcore_axis_name="s")

    @functools.partial(
        pl.kernel,
        out_type=(
            jax.ShapeDtypeStruct((NC, _N, _H), jnp.float32),
            jax.ShapeDtypeStruct((NC, _N, _H), jnp.float32),
            jax.ShapeDtypeStruct((_N, _H), jnp.float32),
        ),
        mesh=mesh,
        scratch_types=[
            pltpu.VMEM_SHARED((_N, _H), jnp.float32),
            pltpu.VMEM((_CH,), jnp.int32),
            pltpu.VMEM((_CH,), jnp.int32),
            pltpu.VMEM((_CH, _H), jnp.float32),
            pltpu.VMEM((_ZR, _H), jnp.float32),
            pltpu.VMEM((_ZR, _H), jnp.float32),
            pltpu.VMEM((_VCH,), jnp.int32),
            pltpu.VMEM((_VCH, _H), jnp.float32),
            pltpu.SemaphoreType.DMA,
        ],
    )
    def k(ea_hbm, dst_hbm, eae_hbm, xemb_hbm, vemb_hbm, temb_hbm,
          attr_out, emb_out, veb_out,
          acc, didx, eidx, rows, zb, bb, nidx, vrows, sem):
        core = lax.axis_index("c")
        sid = lax.axis_index("s")
        wid = sid * NC + core

        def zero_acc():
            for q in range(_ZQ):
                zc = sid + NS * q

                @pl.when(zc < _ZNCHUNK)
                def _():
                    pltpu.sync_copy(zb, acc.at[pl.ds(zc * _ZR, _ZR)])

        def copy_out(dst):
            for q in range(_ZQ):
                zc = sid + NS * q

                @pl.when(zc < _ZNCHUNK)
                def _():
                    r0 = zc * _ZR
                    pltpu.sync_copy(acc.at[pl.ds(r0, _ZR)], bb)
                    pltpu.sync_copy(bb, dst.at[core, pl.ds(r0, _ZR)])

        _zero_2d(zb, _ZR, _H)
        zero_acc()
        plsc.subcore_barrier()

        def ebody(j, _):
            cid = wid + NW * j

            @pl.when(cid < _NCHUNK)
            def _():
                pltpu.sync_copy(dst_hbm.at[pl.ds(cid * _CH, _CH)], didx)
                pltpu.sync_copy(ea_hbm.at[pl.ds(cid * _CH, _CH)], rows)
                pltpu.sync_copy(rows, acc.at[didx], add=True)

            return 0

        lax.fori_loop(0, _JMAX, ebody, 0)

        plsc.subcore_barrier()
        copy_out(attr_out)
        zero_acc()
        plsc.subcore_barrier()

        def ebody2(j, _):
            cid = wid + NW * j

            @pl.when(cid < _NCHUNK)
            def _():
                pltpu.sync_copy(dst_hbm.at[pl.ds(cid * _CH, _CH)], didx)
                pltpu.sync_copy(eae_hbm.at[pl.ds(cid * _CH, _CH)], eidx)
                pltpu.async_copy(temb_hbm.at[eidx], rows, sem).wait()
                pltpu.sync_copy(rows, acc.at[didx], add=True)

            return 0

        lax.fori_loop(0, _JMAX, ebody2, 0)

        def vbody(j, _):
            cid = wid + NW * j

            @pl.when(cid < _VNCHUNK)
            def _():
                pltpu.sync_copy(xemb_hbm.at[pl.ds(cid * _VCH, _VCH)], nidx)
                pltpu.async_copy(vemb_hbm.at[nidx], vrows, sem).wait()
                pltpu.sync_copy(vrows, veb_out.at[pl.ds(cid * _VCH, _VCH)])

            return 0

        lax.fori_loop(0, _VJ, vbody, 0)

        plsc.subcore_barrier()
        copy_out(emb_out)

    return k


def _make_spmv():
    mesh = plsc.VectorSubcoreMesh(core_axis_name="c", subcore_axis_name="s")

    @functools.partial(
        pl.kernel,
        out_type=jax.ShapeDtypeStruct((NC, _N, _H), jnp.float32),
        mesh=mesh,
        scratch_types=[
            pltpu.VMEM_SHARED((_N, _H), jnp.float32),
            pltpu.VMEM((_CH,), jnp.int32),
            pltpu.VMEM((_CH,), jnp.int32),
            pltpu.VMEM((_CH, _H), jnp.float32),
            pltpu.VMEM((_ZR, _H), jnp.float32),
            pltpu.SemaphoreType.DMA,
        ],
    )
    def k(h_hbm, src_hbm, dst_hbm, part_out, acc, sidx, didx, rows, zb, sem):
        core = lax.axis_index("c")
        sid = lax.axis_index("s")
        wid = sid * NC + core

        _zero_2d(zb, _ZR, _H)
        for q in range(_ZQ):
            zc = sid + NS * q

            @pl.when(zc < _ZNCHUNK)
            def _():
                pltpu.sync_copy(zb, acc.at[pl.ds(zc * _ZR, _ZR)])

        plsc.subcore_barrier()

        def ebody(j, _):
            cid = wid + NW * j

            @pl.when(cid < _NCHUNK)
            def _():
                pltpu.sync_copy(src_hbm.at[pl.ds(cid * _CH, _CH)], sidx)
                pltpu.sync_copy(dst_hbm.at[pl.ds(cid * _CH, _CH)], didx)
                pltpu.async_copy(h_hbm.at[sidx], rows, sem).wait()
                pltpu.sync_copy(rows, acc.at[didx], add=True)

            return 0

        lax.fori_loop(0, _JMAX, ebody, 0)

        plsc.subcore_barrier()
        for q in range(_ZQ):
            zc = sid + NS * q

            @pl.when(zc < _ZNCHUNK)
            def _():
                r0 = zc * _ZR
                pltpu.sync_copy(acc.at[pl.ds(r0, _ZR)], zb)
                pltpu.sync_copy(zb, part_out.at[core, pl.ds(r0, _ZR)])

    return k


def _dot(a, b):
    return jnp.dot(a, b, preferred_element_type=jnp.float32)


def _pre_body(x_ref, veb_ref, ap_ref, ep_ref, wv_ref, we_ref,
              h0_ref, eagg_ref):
    h0_ref[...] = _dot(x_ref[...], wv_ref[...]) + veb_ref[...]
    attr = ap_ref[0] + ap_ref[1]
    eagg_ref[...] = _dot(attr, we_ref[...]) + ep_ref[0] + ep_ref[1]


def _layer_body(h_ref, part_ref, eagg_ref, wlin_ref, wr_ref, wroot_ref,
                b_ref, g_ref, be_ref, wres_ref, out_ref):
    h = h_ref[...]
    agg = part_ref[0] + part_ref[1] + _dot(eagg_ref[...], wlin_ref[...])
    out = _dot(agg, wr_ref[...]) + _dot(h, wroot_ref[...]) + b_ref[...]
    out = jnp.maximum(out, 0.0)
    mean = jnp.mean(out, axis=0, keepdims=True)
    ctr = out - mean
    var = jnp.mean(ctr * ctr, axis=0, keepdims=True)
    outn = ctr * lax.rsqrt(var + 1e-5) * g_ref[...] + be_ref[...]
    out_ref[...] = outn + _dot(h, wres_ref[...])


def _head_body(h_ref, w1_ref, b1_ref, w2_ref, b2_ref, y_ref):
    t = _dot(h_ref[...], w1_ref[...]) + b1_ref[...]
    t = t * 0.5 * (1.0 + lax.erf(t * (2.0 ** -0.5)))
    y_ref[...] = _dot(t, w2_ref[...]) + b2_ref[...]


def kernel(x, x_emb, edge_index, edge_attr, edge_attr_emb, vert_ff_w,
           vert_emb_t, edge_ff_w, edge_emb_t, edge_lin_w, conv_rel_w,
           conv_root_w, conv_b, bn_gamma, bn_beta, res_lin_w, head_w1,
           head_b1, head_w2, head_b2):
    src1d = edge_index[0]
    dst1d = edge_index[1]
    eae1d = edge_attr_emb.astype(jnp.int32)

    attr_part, emb_part, veb = _make_edge_pre()(
        edge_attr, dst1d, eae1d, x_emb.astype(jnp.int32), vert_emb_t,
        edge_emb_t)

    h0, ea_agg = pl.pallas_call(
        _pre_body,
        out_shape=(
            jax.ShapeDtypeStruct((_N, _H), jnp.float32),
            jax.ShapeDtypeStruct((_N, _H), jnp.float32),
        ),
    )(x, veb, attr_part, emb_part, vert_ff_w, edge_ff_w)

    spmv = _make_spmv()
    layer = pl.pallas_call(
        _layer_body,
        out_shape=jax.ShapeDtypeStruct((_N, _H), jnp.float32),
    )

    h = h0
    for l in range(_L):
        part = spmv(h, src1d, dst1d)
        h = layer(h, part, ea_agg, edge_lin_w[l], conv_rel_w[l],
                  conv_root_w[l], conv_b[l].reshape(1, _H),
                  bn_gamma[l].reshape(1, _H), bn_beta[l].reshape(1, _H),
                  res_lin_w[l])

    y = pl.pallas_call(
        _head_body,
        out_shape=jax.ShapeDtypeStruct((_N, 1), jnp.float32),
    )(h, head_w1, head_b1.reshape(1, -1), head_w2, head_b2.reshape(1, 1))
    return y
